# baseline (speedup 1.0000x reference)
import functools

import jax
import jax.numpy as jnp
from jax import lax
from jax.experimental import pallas as pl
from jax.experimental.pallas import tpu as pltpu
from jax.experimental.pallas import tpu_sc as plsc

_IDX_BLK = 128


@functools.lru_cache(maxsize=None)
def _build(V, D, N):
    info = plsc.get_sparse_core_info()
    NC, NS = info.num_cores, info.num_subcores
    NW = NC * NS
    assert N % (NW * _IDX_BLK) == 0
    n_per_w = N // NW
    NB = n_per_w // _IDX_BLK

    mesh = plsc.VectorSubcoreMesh(core_axis_name="c", subcore_axis_name="s")

    @functools.partial(
        pl.kernel,
        mesh=mesh,
        compiler_params=pltpu.CompilerParams(use_tc_tiling_on_sc=False),
        out_type=jax.ShapeDtypeStruct((N, D), jnp.float32),
        scratch_types=[
            pltpu.VMEM((NB, _IDX_BLK), jnp.int32),
            pltpu.VMEM((_IDX_BLK, D), jnp.float32),
            pltpu.SemaphoreType.DMA,
        ],
    )
    def k(table_hbm, idx_hbm, out_hbm, idx_v, rows_v, gsem):
        wid = lax.axis_index("s") * NC + lax.axis_index("c")
        base = wid * n_per_w
        pltpu.sync_copy(idx_hbm.at[wid], idx_v)

        def body(j, carry):
            pltpu.async_copy(table_hbm.at[idx_v.at[j]], rows_v, gsem).wait()
            pltpu.sync_copy(rows_v, out_hbm.at[pl.ds(base + j * _IDX_BLK, _IDX_BLK)])
            return carry

        lax.fori_loop(0, NB, body, 0)

    return k, NW, NB


def kernel(table, x):
    V, D = table.shape
    orig_shape = x.shape
    N = x.size
    k, NW, NB = _build(V, D, N)
    idx = x.reshape(NW, NB, _IDX_BLK).astype(jnp.int32)
    out = k(table, idx)
    return out.reshape(*orig_shape, D)


# speedup vs baseline: 1.1161x; 1.1161x over previous
import functools

import jax
import jax.numpy as jnp
from jax import lax
from jax.experimental import pallas as pl
from jax.experimental.pallas import tpu as pltpu
from jax.experimental.pallas import tpu_sc as plsc

_IDX_BLK = 128
_NBUF = 8


@functools.lru_cache(maxsize=None)
def _build(V, D, N):
    info = plsc.get_sparse_core_info()
    NC, NS = info.num_cores, info.num_subcores
    NW = NC * NS
    assert N % (NW * _IDX_BLK) == 0
    n_per_w = N // NW
    NB = n_per_w // _IDX_BLK
    assert NB % _NBUF == 0

    mesh = plsc.VectorSubcoreMesh(core_axis_name="c", subcore_axis_name="s")

    @functools.partial(
        pl.kernel,
        mesh=mesh,
        compiler_params=pltpu.CompilerParams(use_tc_tiling_on_sc=False),
        out_type=jax.ShapeDtypeStruct((N, D), jnp.float32),
        scratch_types=[
            pltpu.VMEM((NB, _IDX_BLK), jnp.int32),
            pltpu.VMEM((_NBUF, _IDX_BLK, D), jnp.float32),
            [pltpu.SemaphoreType.DMA] * _NBUF,
            [pltpu.SemaphoreType.DMA] * _NBUF,
        ],
    )
    def k(table_hbm, idx_hbm, out_hbm, idx_v, rows_v, gsems, ssems):
        wid = lax.axis_index("s") * NC + lax.axis_index("c")
        base = wid * n_per_w
        pltpu.sync_copy(idx_hbm.at[wid], idx_v)

        def gather(g, b):
            return pltpu.make_async_copy(
                table_hbm.at[idx_v.at[g]], rows_v.at[b], gsems[b])

        def store(g, b):
            return pltpu.make_async_copy(
                rows_v.at[b],
                out_hbm.at[pl.ds(base + g * _IDX_BLK, _IDX_BLK)],
                ssems[b])

        for b in range(_NBUF):
            gather(b, b).start()

        def group(i, carry):
            o = i * _NBUF
            for b in range(_NBUF):
                gather(o + b, b).wait()
                store(o + b, b).start()
            for b in range(_NBUF):
                store(o + b, b).wait()
                gather(o + _NBUF + b, b).start()
            return carry

        lax.fori_loop(0, NB // _NBUF - 1, group, 0)

        o = NB - _NBUF
        for b in range(_NBUF):
            gather(o + b, b).wait()
            store(o + b, b).start()
        for b in range(_NBUF):
            store(o + b, b).wait()

    return k, NW, NB


def kernel(table, x):
    V, D = table.shape
    orig_shape = x.shape
    N = x.size
    k, NW, NB = _build(V, D, N)
    idx = x.reshape(NW, NB, _IDX_BLK).astype(jnp.int32)
    out = k(table, idx)
    return out.reshape(*orig_shape, D)
